# Initial kernel scaffold; baseline (speedup 1.0000x reference)
import jax
import jax.numpy as jnp
from jax.experimental import pallas as pl


def kernel(decoded_last_hidden, decoded_probs, target, optor_keys, optor_vals, const_keys, const_vals):
    raise NotImplementedError("write your pallas kernel here")


# speedup vs baseline: 1.2821x; 1.2821x over previous
import jax
import jax.numpy as jnp
from jax.experimental import pallas as pl
from jax.experimental.pallas import tpu as pltpu

_K = 32
_B, _T, _D, _V, _N = 64, 4, 1024, 32000, 100000
_Q = _B * _T
_C = 1000
_NCHUNK = _N // _C


def _topk_body(qn_ref, q_ref, keys_ref, kn_ref, bd_ref, bi_ref):
    step = pl.program_id(0)

    @pl.when(step == 0)
    def _init():
        bd_ref[...] = jnp.full((_Q, _K), jnp.inf, jnp.float32)
        bi_ref[...] = jnp.zeros((_Q, _K), jnp.int32)

    q = q_ref[...]
    keys = keys_ref[...]
    kn = kn_ref[0]
    qn = qn_ref[...]
    dot = jax.lax.dot_general(q, keys, (((1,), (1,)), ((), ())),
                              preferred_element_type=jnp.float32)
    dists = qn - 2.0 * dot + kn
    base = step * _C
    gidx = base + jax.lax.broadcasted_iota(jnp.int32, (_Q, _C), 1)

    cd = jnp.concatenate([bd_ref[...], dists], axis=1)
    ci = jnp.concatenate([bi_ref[...], gidx], axis=1)
    w = _K + _C
    col = jax.lax.broadcasted_iota(jnp.int32, (_Q, w), 1)
    nd, ni = [], []
    for _ in range(_K):
        m = jnp.min(cd, axis=1, keepdims=True)
        pos = jnp.min(jnp.where(cd == m, col, w), axis=1, keepdims=True)
        sel = col == pos
        vi = jnp.sum(jnp.where(sel, ci, 0), axis=1, keepdims=True)
        nd.append(m)
        ni.append(vi)
        cd = jnp.where(sel, jnp.inf, cd)
    bd_ref[...] = jnp.concatenate(nd, axis=1)
    bi_ref[...] = jnp.concatenate(ni, axis=1)


def _store_topk(qn, q, keys, kn, interpret=False):
    kn3 = kn.reshape(_NCHUNK, 1, _C)
    bd, bi = pl.pallas_call(
        _topk_body,
        grid=(_NCHUNK,),
        in_specs=[
            pl.BlockSpec((_Q, 1), lambda i: (0, 0)),
            pl.BlockSpec((_Q, _D), lambda i: (0, 0)),
            pl.BlockSpec((_C, _D), lambda i: (i, 0)),
            pl.BlockSpec((1, 1, _C), lambda i: (i, 0, 0)),
        ],
        out_specs=[
            pl.BlockSpec((_Q, _K), lambda i: (0, 0)),
            pl.BlockSpec((_Q, _K), lambda i: (0, 0)),
        ],
        out_shape=[
            jax.ShapeDtypeStruct((_Q, _K), jnp.float32),
            jax.ShapeDtypeStruct((_Q, _K), jnp.int32),
        ],
        compiler_params=pltpu.CompilerParams(
            dimension_semantics=("arbitrary",)),
        interpret=interpret,
    )(qn, q, keys, kn3)
    return bd, bi


def kernel(decoded_last_hidden, decoded_probs, target,
           optor_keys, optor_vals, const_keys, const_vals):
    noise = jax.random.normal(jax.random.key(1), decoded_last_hidden.shape,
                              dtype=decoded_last_hidden.dtype)
    h = decoded_last_hidden + noise * 5.0
    q = h.reshape(-1, h.shape[-1])
    qn = jnp.sum(q * q, axis=-1, keepdims=True)
    okn = jnp.sum(optor_keys * optor_keys, axis=-1)
    ckn = jnp.sum(const_keys * const_keys, axis=-1)

    od, oi = _store_topk(qn, q, optor_keys, okn)
    cd, ci = _store_topk(qn, q, const_keys, ckn)

    optor_v = jnp.take(optor_vals, oi, axis=0).reshape(_B, _T, _K)
    const_v = jnp.take(const_vals, ci, axis=0).reshape(_B, _T, _K)
    optor_d = od.reshape(_B, _T, _K)
    const_d = cd.reshape(_B, _T, _K)
    return (decoded_probs, h, target, optor_v, optor_d, const_v, const_d)


# speedup vs baseline: 2.5862x; 2.0171x over previous
import jax
import jax.numpy as jnp
from jax.experimental import pallas as pl
from jax.experimental.pallas import tpu as pltpu

_K = 32
_B, _T, _D, _V, _N = 64, 4, 1024, 32000, 100000
_Q = _B * _T
_C = 1000
_NCHUNK = _N // _C


def _topk_body(qn_ref, q_ref, keys_ref, kn_ref, bd_ref, bi_ref):
    step = pl.program_id(0)

    q = q_ref[...]
    keys = keys_ref[...]
    kn = kn_ref[0]
    qn = qn_ref[...]
    dot = jax.lax.dot_general(q, keys, (((1,), (1,)), ((), ())),
                              preferred_element_type=jnp.float32)
    dists = qn - 2.0 * dot + kn
    base = step * _C
    gidx = base + jax.lax.broadcasted_iota(jnp.int32, (_Q, _C), 1)
    col = jax.lax.broadcasted_iota(jnp.int32, (_Q, _C), 1)

    @pl.when(step == 0)
    def _first():
        cd = dists
        nd, ni = [], []
        for _ in range(_K):
            m = jnp.min(cd, axis=1, keepdims=True)
            pos = jnp.min(jnp.where(cd == m, col, _C), axis=1, keepdims=True)
            sel = col == pos
            vi = jnp.sum(jnp.where(sel, gidx, 0), axis=1, keepdims=True)
            nd.append(m)
            ni.append(vi)
            cd = jnp.where(sel, jnp.inf, cd)
        bd_ref[...] = jnp.concatenate(nd, axis=1)
        bi_ref[...] = jnp.concatenate(ni, axis=1)

    @pl.when(step > 0)
    def _rest():
        bd0 = bd_ref[...]
        bi0 = bi_ref[...]
        tau = bd0[:, _K - 1:_K]
        beats = dists < tau
        cmax = jnp.max(jnp.sum(beats.astype(jnp.int32), axis=1))
        cd0 = jnp.where(beats, dists, jnp.inf)
        kcol = jax.lax.broadcasted_iota(jnp.int32, (_Q, _K), 1)

        def body(_, carry):
            cd, bd, bi = carry
            m = jnp.min(cd, axis=1, keepdims=True)
            pos = jnp.min(jnp.where(cd == m, col, _C), axis=1, keepdims=True)
            sel = col == pos
            vi = jnp.sum(jnp.where(sel, gidx, 0), axis=1, keepdims=True)
            cd = jnp.where(sel, jnp.inf, cd)
            ins = m < bd[:, _K - 1:_K]
            posb = jnp.sum((bd <= m).astype(jnp.int32), axis=1, keepdims=True)
            shd = jnp.roll(bd, 1, axis=1)
            shi = jnp.roll(bi, 1, axis=1)
            nbd = jnp.where(kcol < posb, bd, jnp.where(kcol == posb, m, shd))
            nbi = jnp.where(kcol < posb, bi, jnp.where(kcol == posb, vi, shi))
            bd = jnp.where(ins, nbd, bd)
            bi = jnp.where(ins, nbi, bi)
            return cd, bd, bi

        _, bdf, bif = jax.lax.fori_loop(0, cmax, body, (cd0, bd0, bi0))
        bd_ref[...] = bdf
        bi_ref[...] = bif


def _store_topk(qn, q, keys, kn, interpret=False):
    kn3 = kn.reshape(_NCHUNK, 1, _C)
    bd, bi = pl.pallas_call(
        _topk_body,
        grid=(_NCHUNK,),
        in_specs=[
            pl.BlockSpec((_Q, 1), lambda i: (0, 0)),
            pl.BlockSpec((_Q, _D), lambda i: (0, 0)),
            pl.BlockSpec((_C, _D), lambda i: (i, 0)),
            pl.BlockSpec((1, 1, _C), lambda i: (i, 0, 0)),
        ],
        out_specs=[
            pl.BlockSpec((_Q, _K), lambda i: (0, 0)),
            pl.BlockSpec((_Q, _K), lambda i: (0, 0)),
        ],
        out_shape=[
            jax.ShapeDtypeStruct((_Q, _K), jnp.float32),
            jax.ShapeDtypeStruct((_Q, _K), jnp.int32),
        ],
        compiler_params=pltpu.CompilerParams(
            dimension_semantics=("arbitrary",)),
        interpret=interpret,
    )(qn, q, keys, kn3)
    return bd, bi


def kernel(decoded_last_hidden, decoded_probs, target,
           optor_keys, optor_vals, const_keys, const_vals):
    noise = jax.random.normal(jax.random.key(1), decoded_last_hidden.shape,
                              dtype=decoded_last_hidden.dtype)
    h = decoded_last_hidden + noise * 5.0
    q = h.reshape(-1, h.shape[-1])
    qn = jnp.sum(q * q, axis=-1, keepdims=True)
    okn = jnp.sum(optor_keys * optor_keys, axis=-1)
    ckn = jnp.sum(const_keys * const_keys, axis=-1)

    od, oi = _store_topk(qn, q, optor_keys, okn)
    cd, ci = _store_topk(qn, q, const_keys, ckn)

    optor_v = jnp.take(optor_vals, oi, axis=0).reshape(_B, _T, _K)
    const_v = jnp.take(const_vals, ci, axis=0).reshape(_B, _T, _K)
    optor_d = od.reshape(_B, _T, _K)
    const_d = cd.reshape(_B, _T, _K)
    return (decoded_probs, h, target, optor_v, optor_d, const_v, const_d)


# speedup vs baseline: 3.1329x; 1.2114x over previous
import jax
import jax.numpy as jnp
from jax.experimental import pallas as pl
from jax.experimental.pallas import tpu as pltpu

_K = 32
_B, _T, _D, _V, _N = 64, 4, 1024, 32000, 100000
_Q = _B * _T
_C = 1000
_NCHUNK = _N // _C


def _topk_body(qn_ref, q_ref, keys_ref, kn_ref, bd_ref, bi_ref):
    step = pl.program_id(0)

    q = q_ref[...]
    keys = keys_ref[...]
    kn = kn_ref[0]
    qn = qn_ref[...]
    dot = jax.lax.dot_general(q, keys, (((1,), (1,)), ((), ())),
                              preferred_element_type=jnp.float32)
    dists = qn - 2.0 * dot + kn
    base = step * _C
    colf = jax.lax.broadcasted_iota(jnp.int32, (_Q, _C), 1).astype(jnp.float32)

    @pl.when(step == 0)
    def _first():
        cd = dists
        nd, ni = [], []
        for _ in range(_K):
            m = jnp.min(cd, axis=1, keepdims=True)
            posf = jnp.min(jnp.where(cd == m, colf, 1e9),
                           axis=1, keepdims=True)
            nd.append(m)
            ni.append(base + posf.astype(jnp.int32))
            cd = jnp.where(colf == posf, jnp.inf, cd)
        bd_ref[...] = jnp.concatenate(nd, axis=1)
        bi_ref[...] = jnp.concatenate(ni, axis=1)

    @pl.when(step > 0)
    def _rest():
        bd0 = bd_ref[...]
        bi0 = bi_ref[...]
        tau = bd0[:, _K - 1:_K]
        beats = dists < tau
        cmax = jnp.max(jnp.sum(beats.astype(jnp.int32), axis=1))
        cd0 = jnp.where(beats, dists, jnp.inf)
        kcol = jax.lax.broadcasted_iota(jnp.int32, (_Q, _K), 1)

        def body(_, carry):
            cd, bd, bi = carry
            m = jnp.min(cd, axis=1, keepdims=True)
            posf = jnp.min(jnp.where(cd == m, colf, 1e9),
                           axis=1, keepdims=True)
            vi = base + posf.astype(jnp.int32)
            cd = jnp.where(colf == posf, jnp.inf, cd)
            ins = m < bd[:, _K - 1:_K]
            posb = jnp.sum((bd <= m).astype(jnp.int32), axis=1, keepdims=True)
            shd = jnp.roll(bd, 1, axis=1)
            shi = jnp.roll(bi, 1, axis=1)
            nbd = jnp.where(kcol < posb, bd, jnp.where(kcol == posb, m, shd))
            nbi = jnp.where(kcol < posb, bi, jnp.where(kcol == posb, vi, shi))
            bd = jnp.where(ins, nbd, bd)
            bi = jnp.where(ins, nbi, bi)
            return cd, bd, bi

        _, bdf, bif = jax.lax.fori_loop(0, cmax, body, (cd0, bd0, bi0))
        bd_ref[...] = bdf
        bi_ref[...] = bif


def _store_topk(qn, q, keys, kn, interpret=False):
    kn3 = kn.reshape(_NCHUNK, 1, _C)
    bd, bi = pl.pallas_call(
        _topk_body,
        grid=(_NCHUNK,),
        in_specs=[
            pl.BlockSpec((_Q, 1), lambda i: (0, 0)),
            pl.BlockSpec((_Q, _D), lambda i: (0, 0)),
            pl.BlockSpec((_C, _D), lambda i: (i, 0)),
            pl.BlockSpec((1, 1, _C), lambda i: (i, 0, 0)),
        ],
        out_specs=[
            pl.BlockSpec((_Q, _K), lambda i: (0, 0)),
            pl.BlockSpec((_Q, _K), lambda i: (0, 0)),
        ],
        out_shape=[
            jax.ShapeDtypeStruct((_Q, _K), jnp.float32),
            jax.ShapeDtypeStruct((_Q, _K), jnp.int32),
        ],
        compiler_params=pltpu.CompilerParams(
            dimension_semantics=("arbitrary",)),
        interpret=interpret,
    )(qn, q, keys, kn3)
    return bd, bi


def kernel(decoded_last_hidden, decoded_probs, target,
           optor_keys, optor_vals, const_keys, const_vals):
    noise = jax.random.normal(jax.random.key(1), decoded_last_hidden.shape,
                              dtype=decoded_last_hidden.dtype)
    h = decoded_last_hidden + noise * 5.0
    q = h.reshape(-1, h.shape[-1])
    qn = jnp.sum(q * q, axis=-1, keepdims=True)
    okn = jnp.sum(optor_keys * optor_keys, axis=-1)
    ckn = jnp.sum(const_keys * const_keys, axis=-1)

    od, oi = _store_topk(qn, q, optor_keys, okn)
    cd, ci = _store_topk(qn, q, const_keys, ckn)

    optor_v = jnp.take(optor_vals, oi, axis=0).reshape(_B, _T, _K)
    const_v = jnp.take(const_vals, ci, axis=0).reshape(_B, _T, _K)
    optor_d = od.reshape(_B, _T, _K)
    const_d = cd.reshape(_B, _T, _K)
    return (decoded_probs, h, target, optor_v, optor_d, const_v, const_d)


# speedup vs baseline: 3.5417x; 1.1305x over previous
import jax
import jax.numpy as jnp
from jax.experimental import pallas as pl
from jax.experimental.pallas import tpu as pltpu

_K = 32
_B, _T, _D, _V, _N = 64, 4, 1024, 32000, 100000
_Q = _B * _T
_C = 1000
_NCHUNK = _N // _C


def _topk_body(qn_ref, q_ref, keys_ref, kn_ref, bd_ref, bi_ref, cd_ref):
    step = pl.program_id(0)

    q = q_ref[...]
    keys = keys_ref[...]
    kn = kn_ref[0]
    qn = qn_ref[...]
    dot = jax.lax.dot_general(q, keys, (((1,), (1,)), ((), ())),
                              preferred_element_type=jnp.float32)
    dists = qn - 2.0 * dot + kn
    base = step * _C
    colf = jax.lax.broadcasted_iota(jnp.int32, (_Q, _C), 1).astype(jnp.float32)

    @pl.when(step == 0)
    def _first():
        cd_ref[...] = dists
        nd, ni = [], []
        for _ in range(_K):
            cd = cd_ref[...]
            m = jnp.min(cd, axis=1, keepdims=True)
            posf = jnp.min(jnp.where(cd == m, colf, 1e9),
                           axis=1, keepdims=True)
            nd.append(m)
            ni.append(base + posf.astype(jnp.int32))
            cd_ref[...] = jnp.where(colf == posf, jnp.inf, cd)
        bd_ref[...] = jnp.concatenate(nd, axis=1)
        bi_ref[...] = jnp.concatenate(ni, axis=1)

    @pl.when(step > 0)
    def _rest():
        tau = bd_ref[:, _K - 1:_K]
        any_beats = jnp.any(dists < tau)
        kcol = jax.lax.broadcasted_iota(jnp.int32, (_Q, _K), 1)

        @pl.when(any_beats)
        def _merge():
            cd_ref[...] = dists

            def body(_):
                cd = cd_ref[...]
                bd = bd_ref[...]
                bi = bi_ref[...]
                m = jnp.min(cd, axis=1, keepdims=True)
                posf = jnp.min(jnp.where(cd == m, colf, 1e9),
                               axis=1, keepdims=True)
                vi = base + posf.astype(jnp.int32)
                cd_ref[...] = jnp.where(colf == posf, jnp.inf, cd)
                ins = m < bd[:, _K - 1:_K]
                posb = jnp.sum((bd <= m).astype(jnp.int32),
                               axis=1, keepdims=True)
                shd = jnp.roll(bd, 1, axis=1)
                shi = jnp.roll(bi, 1, axis=1)
                nbd = jnp.where(kcol < posb, bd,
                                jnp.where(kcol == posb, m, shd))
                nbi = jnp.where(kcol < posb, bi,
                                jnp.where(kcol == posb, vi, shi))
                bd_ref[...] = jnp.where(ins, nbd, bd)
                bi_ref[...] = jnp.where(ins, nbi, bi)
                return jnp.any(ins)

            jax.lax.while_loop(lambda go: go, body, any_beats)


def _store_topk(qn, q, keys, kn, interpret=False):
    kn3 = kn.reshape(_NCHUNK, 1, _C)
    bd, bi = pl.pallas_call(
        _topk_body,
        grid=(_NCHUNK,),
        in_specs=[
            pl.BlockSpec((_Q, 1), lambda i: (0, 0)),
            pl.BlockSpec((_Q, _D), lambda i: (0, 0)),
            pl.BlockSpec((_C, _D), lambda i: (i, 0)),
            pl.BlockSpec((1, 1, _C), lambda i: (i, 0, 0)),
        ],
        out_specs=[
            pl.BlockSpec((_Q, _K), lambda i: (0, 0)),
            pl.BlockSpec((_Q, _K), lambda i: (0, 0)),
        ],
        out_shape=[
            jax.ShapeDtypeStruct((_Q, _K), jnp.float32),
            jax.ShapeDtypeStruct((_Q, _K), jnp.int32),
        ],
        compiler_params=pltpu.CompilerParams(
            dimension_semantics=("arbitrary",)),
        scratch_shapes=[pltpu.VMEM((_Q, _C), jnp.float32)],
        interpret=interpret,
    )(qn, q, keys, kn3)
    return bd, bi


def kernel(decoded_last_hidden, decoded_probs, target,
           optor_keys, optor_vals, const_keys, const_vals):
    noise = jax.random.normal(jax.random.key(1), decoded_last_hidden.shape,
                              dtype=decoded_last_hidden.dtype)
    h = decoded_last_hidden + noise * 5.0
    q = h.reshape(-1, h.shape[-1])
    qn = jnp.sum(q * q, axis=-1, keepdims=True)
    okn = jnp.sum(optor_keys * optor_keys, axis=-1)
    ckn = jnp.sum(const_keys * const_keys, axis=-1)

    od, oi = _store_topk(qn, q, optor_keys, okn)
    cd, ci = _store_topk(qn, q, const_keys, ckn)

    optor_v = jnp.take(optor_vals, oi, axis=0).reshape(_B, _T, _K)
    const_v = jnp.take(const_vals, ci, axis=0).reshape(_B, _T, _K)
    optor_d = od.reshape(_B, _T, _K)
    const_d = cd.reshape(_B, _T, _K)
    return (decoded_probs, h, target, optor_v, optor_d, const_v, const_d)
